# baseline (speedup 1.0000x reference)
import functools

import jax
import jax.numpy as jnp
from jax import lax
from jax.experimental import pallas as pl
from jax.experimental.pallas import tpu as pltpu
from jax.experimental.pallas import tpu_sc as plsc

N = 10000
E = 320000
D = 128
NC, NS = 2, 16
NW = NC * NS
EPW = E // NW
CH = 80
NIT = EPW // CH
NIT2 = 126
NPAD = 10240
RPS = NPAD // NS
TRASH = NPAD - 1

_mesh = plsc.VectorSubcoreMesh(core_axis_name="c", subcore_axis_name="s")


@functools.partial(
    pl.kernel,
    mesh=_mesh,
    out_type=jax.ShapeDtypeStruct((NC, NPAD, D), jnp.float32),
    scratch_types=[
        pltpu.VMEM((2, CH), jnp.int32),
        pltpu.VMEM((2, CH), jnp.int32),
        pltpu.VMEM((CH,), jnp.int32),
        pltpu.VMEM((2, CH, D), jnp.float32),
        pltpu.VMEM_SHARED((NPAD, D), jnp.float32),
        [pltpu.SemaphoreType.DMA for _ in range(2)],
        [pltpu.SemaphoreType.DMA for _ in range(2)],
        pltpu.SemaphoreType.DMA,
    ],
)
def _sum_agg(h_hbm, srcw_hbm, dstw_hbm, iota_hbm, z_d_hbm,
             psums_hbm, src_v, dst_v, own_v, rows_v, acc_s, gsems, ssems,
             sem):
    c = lax.axis_index("c")
    s = lax.axis_index("s")
    rbase = s * RPS
    wid = s * NC + c

    pltpu.sync_copy(z_d_hbm, rows_v.at[0])

    def zstep(k, carry):
        pltpu.sync_copy(iota_hbm.at[pl.ds(rbase + k * CH, CH)], own_v)
        pltpu.sync_copy(rows_v.at[0], acc_s.at[own_v])
        return carry

    lax.fori_loop(0, RPS // CH, zstep, 0)
    plsc.subcore_barrier()

    for b in range(2):
        pltpu.sync_copy(srcw_hbm.at[wid, b], src_v.at[b])
        pltpu.sync_copy(dstw_hbm.at[wid, b], dst_v.at[b])
        pltpu.async_copy(h_hbm.at[src_v.at[b]], rows_v.at[b], gsems[b])

    def step(j, carry):
        i0 = j * 2
        for b in range(2):
            pltpu.make_async_copy(
                h_hbm.at[src_v.at[b]], rows_v.at[b], gsems[b]).wait()
            pltpu.async_copy(rows_v.at[b], acc_s.at[dst_v.at[b]], ssems[b],
                             add=True)
        for b in range(2):
            pltpu.make_async_copy(
                rows_v.at[b], acc_s.at[dst_v.at[b]], ssems[b]).wait()
            pltpu.sync_copy(srcw_hbm.at[wid, i0 + b + 2], src_v.at[b])
            pltpu.sync_copy(dstw_hbm.at[wid, i0 + b + 2], dst_v.at[b])
            pltpu.async_copy(h_hbm.at[src_v.at[b]], rows_v.at[b], gsems[b])
        return carry

    lax.fori_loop(0, NIT2 // 2, step, 0)
    for b in range(2):
        pltpu.make_async_copy(
            h_hbm.at[src_v.at[b]], rows_v.at[b], gsems[b]).wait()
    plsc.subcore_barrier()

    def wstep(k, carry):
        r = rbase + k * CH
        pltpu.sync_copy(iota_hbm.at[pl.ds(r, CH)], own_v)
        pltpu.async_copy(acc_s.at[own_v], rows_v.at[0], sem).wait()
        pltpu.sync_copy(rows_v.at[0], psums_hbm.at[c, pl.ds(r, CH)])
        return carry

    lax.fori_loop(0, RPS // CH, wstep, 0)


@functools.partial(
    pl.kernel,
    mesh=_mesh,
    out_type=jax.ShapeDtypeStruct((NC, NPAD, D), jnp.float32),
    scratch_types=[
        pltpu.VMEM((CH,), jnp.int32),
        pltpu.VMEM((CH,), jnp.int32),
        pltpu.VMEM((CH, D), jnp.float32),
        pltpu.VMEM((CH, D), jnp.float32),
        pltpu.VMEM_SHARED((NPAD, D), jnp.float32),
        pltpu.SemaphoreType.DMA,
    ],
)
def _cnt_agg(dst_hbm, iota_hbm, z_d_hbm, ones_hbm,
             pcnts_hbm, dst_v, own_v, rows_v, ones_v, cnt_s, sem):
    c = lax.axis_index("c")
    s = lax.axis_index("s")
    rbase = s * RPS

    pltpu.sync_copy(z_d_hbm, rows_v)
    pltpu.sync_copy(ones_hbm, ones_v)

    def zstep(k, carry):
        pltpu.sync_copy(iota_hbm.at[pl.ds(rbase + k * CH, CH)], own_v)
        pltpu.sync_copy(rows_v, cnt_s.at[own_v])
        return carry

    lax.fori_loop(0, RPS // CH, zstep, 0)
    plsc.subcore_barrier()

    ebase = (s * NC + c) * EPW

    def step(i, carry):
        off = ebase + i * CH
        pltpu.sync_copy(dst_hbm.at[pl.ds(off, CH)], dst_v)
        pltpu.sync_copy(ones_v, cnt_s.at[dst_v], add=True)
        return carry

    lax.fori_loop(0, NIT, step, 0)
    plsc.subcore_barrier()

    def wstep(k, carry):
        r = rbase + k * CH
        pltpu.sync_copy(iota_hbm.at[pl.ds(r, CH)], own_v)
        pltpu.async_copy(cnt_s.at[own_v], rows_v, sem).wait()
        pltpu.sync_copy(rows_v, pcnts_hbm.at[c, pl.ds(r, CH)])
        return carry

    lax.fori_loop(0, RPS // CH, wstep, 0)


BLK = 1000


def _combine_body(h_ref, s0_ref, s1_ref, c0_ref, c1_ref, o_ref):
    cnt = c0_ref[0][:, 0:1] + c1_ref[0][:, 0:1]
    sums = s0_ref[0] + s1_ref[0]
    o_ref[...] = h_ref[...] - sums / jnp.maximum(cnt, 1.0)


_combine = pl.pallas_call(
    _combine_body,
    grid=(N // BLK,),
    in_specs=[
        pl.BlockSpec((BLK, D), lambda i: (i, 0)),
        pl.BlockSpec((1, BLK, D), lambda i: (0, i, 0)),
        pl.BlockSpec((1, BLK, D), lambda i: (1, i, 0)),
        pl.BlockSpec((1, BLK, D), lambda i: (0, i, 0)),
        pl.BlockSpec((1, BLK, D), lambda i: (1, i, 0)),
    ],
    out_specs=pl.BlockSpec((BLK, D), lambda i: (i, 0)),
    out_shape=jax.ShapeDtypeStruct((N, D), jnp.float32),
)


def kernel(h, edge_index):
    ei = edge_index.astype(jnp.int32)
    src = ei[0]
    dst = ei[1]
    padw = (NIT2 + 2) * CH - EPW
    srcw = jnp.pad(src.reshape(NW, EPW), ((0, 0), (0, padw)),
                   constant_values=0).reshape(NW, NIT2 + 2, CH)
    dstw = jnp.pad(dst.reshape(NW, EPW), ((0, 0), (0, padw)),
                   constant_values=TRASH).reshape(NW, NIT2 + 2, CH)
    iota = jnp.arange(NPAD, dtype=jnp.int32)
    z_d = jnp.zeros((CH, D), jnp.float32)
    ones = jnp.ones((CH, D), jnp.float32)
    psums = _sum_agg(h, srcw, dstw, iota, z_d)
    pcnts = _cnt_agg(dst, iota, z_d, ones)
    return _combine(h, psums, psums, pcnts, pcnts)


# speedup vs baseline: 1.3650x; 1.3650x over previous
import functools

import jax
import jax.numpy as jnp
from jax import lax
from jax.experimental import pallas as pl
from jax.experimental.pallas import tpu as pltpu
from jax.experimental.pallas import tpu_sc as plsc

N = 10000
E = 320000
D = 128
NC, NS = 2, 16
NW = NC * NS
EPW = E // NW
CH = 80
NIT = EPW // CH
NPAD = 10240
RPS = NPAD // NS

_mesh = plsc.VectorSubcoreMesh(core_axis_name="c", subcore_axis_name="s")


@functools.partial(
    pl.kernel,
    mesh=_mesh,
    out_type=jax.ShapeDtypeStruct((NC, NPAD, D), jnp.float32),
    scratch_types=[
        pltpu.VMEM((CH,), jnp.int32),
        pltpu.VMEM((CH,), jnp.int32),
        pltpu.VMEM((CH,), jnp.int32),
        pltpu.VMEM((CH, D), jnp.float32),
        pltpu.VMEM_SHARED((NPAD, D), jnp.float32),
        pltpu.SemaphoreType.DMA,
    ],
)
def _sum_agg(h_hbm, src_hbm, dst_hbm, iota_hbm, z_d_hbm,
             psums_hbm, src_v, dst_v, own_v, rows_v, acc_s, sem):
    c = lax.axis_index("c")
    s = lax.axis_index("s")
    rbase = s * RPS

    pltpu.sync_copy(z_d_hbm, rows_v)

    def zstep(k, carry):
        pltpu.sync_copy(iota_hbm.at[pl.ds(rbase + k * CH, CH)], own_v)
        pltpu.sync_copy(rows_v, acc_s.at[own_v])
        return carry

    lax.fori_loop(0, RPS // CH, zstep, 0)
    plsc.subcore_barrier()

    ebase = (s * NC + c) * EPW

    def step(i, carry):
        off = ebase + i * CH
        pltpu.sync_copy(src_hbm.at[pl.ds(off, CH)], src_v)
        pltpu.sync_copy(dst_hbm.at[pl.ds(off, CH)], dst_v)
        pltpu.async_copy(h_hbm.at[src_v], rows_v, sem).wait()
        pltpu.sync_copy(rows_v, acc_s.at[dst_v], add=True)
        return carry

    lax.fori_loop(0, NIT, step, 0)
    plsc.subcore_barrier()

    def wstep(k, carry):
        r = rbase + k * CH
        pltpu.sync_copy(iota_hbm.at[pl.ds(r, CH)], own_v)
        pltpu.async_copy(acc_s.at[own_v], rows_v, sem).wait()
        pltpu.sync_copy(rows_v, psums_hbm.at[c, pl.ds(r, CH)])
        return carry

    lax.fori_loop(0, RPS // CH, wstep, 0)


@functools.partial(
    pl.kernel,
    mesh=_mesh,
    out_type=jax.ShapeDtypeStruct((NC, NPAD, D), jnp.float32),
    scratch_types=[
        pltpu.VMEM((CH,), jnp.int32),
        pltpu.VMEM((CH,), jnp.int32),
        pltpu.VMEM((CH, D), jnp.float32),
        pltpu.VMEM((CH, D), jnp.float32),
        pltpu.VMEM_SHARED((NPAD, D), jnp.float32),
        pltpu.SemaphoreType.DMA,
    ],
)
def _cnt_agg(dst_hbm, iota_hbm, z_d_hbm, ones_hbm,
             pcnts_hbm, dst_v, own_v, rows_v, ones_v, cnt_s, sem):
    c = lax.axis_index("c")
    s = lax.axis_index("s")
    rbase = s * RPS

    pltpu.sync_copy(z_d_hbm, rows_v)
    pltpu.sync_copy(ones_hbm, ones_v)

    def zstep(k, carry):
        pltpu.sync_copy(iota_hbm.at[pl.ds(rbase + k * CH, CH)], own_v)
        pltpu.sync_copy(rows_v, cnt_s.at[own_v])
        return carry

    lax.fori_loop(0, RPS // CH, zstep, 0)
    plsc.subcore_barrier()

    ebase = (s * NC + c) * EPW

    def step(i, carry):
        off = ebase + i * CH
        pltpu.sync_copy(dst_hbm.at[pl.ds(off, CH)], dst_v)
        pltpu.sync_copy(ones_v, cnt_s.at[dst_v], add=True)
        return carry

    lax.fori_loop(0, NIT, step, 0)
    plsc.subcore_barrier()

    def wstep(k, carry):
        r = rbase + k * CH
        pltpu.sync_copy(iota_hbm.at[pl.ds(r, CH)], own_v)
        pltpu.async_copy(cnt_s.at[own_v], rows_v, sem).wait()
        pltpu.sync_copy(rows_v, pcnts_hbm.at[c, pl.ds(r, CH)])
        return carry

    lax.fori_loop(0, RPS // CH, wstep, 0)


BLK = 2000


def _combine_body(h_ref, s0_ref, s1_ref, c0_ref, c1_ref, o_ref):
    cnt = c0_ref[0][:, 0:1] + c1_ref[0][:, 0:1]
    sums = s0_ref[0] + s1_ref[0]
    o_ref[...] = h_ref[...] - sums / jnp.maximum(cnt, 1.0)


_combine = pl.pallas_call(
    _combine_body,
    grid=(N // BLK,),
    in_specs=[
        pl.BlockSpec((BLK, D), lambda i: (i, 0)),
        pl.BlockSpec((1, BLK, D), lambda i: (0, i, 0)),
        pl.BlockSpec((1, BLK, D), lambda i: (1, i, 0)),
        pl.BlockSpec((1, BLK, D), lambda i: (0, i, 0)),
        pl.BlockSpec((1, BLK, D), lambda i: (1, i, 0)),
    ],
    out_specs=pl.BlockSpec((BLK, D), lambda i: (i, 0)),
    out_shape=jax.ShapeDtypeStruct((N, D), jnp.float32),
)


def kernel(h, edge_index):
    ei = edge_index.astype(jnp.int32)
    src = ei[0]
    dst = ei[1]
    iota = jnp.arange(NPAD, dtype=jnp.int32)
    z_d = jnp.zeros((CH, D), jnp.float32)
    ones = jnp.ones((CH, D), jnp.float32)
    psums = _sum_agg(h, src, dst, iota, z_d)
    pcnts = _cnt_agg(dst, iota, z_d, ones)
    return _combine(h, psums, psums, pcnts, pcnts)
